# Initial kernel scaffold; baseline (speedup 1.0000x reference)
import jax
import jax.numpy as jnp
from jax.experimental import pallas as pl


def kernel(x, edge_index, W1, b1, W2, b2, W3, b3):
    raise NotImplementedError("write your pallas kernel here")


# speedup vs baseline: 4.8614x; 4.8614x over previous
import functools

import jax
import jax.numpy as jnp
from jax import lax
from jax.experimental import pallas as pl
from jax.experimental.pallas import tpu as pltpu
from jax.experimental.pallas import tpu_sc as plsc

_N = 10000
_E = 160000
_NPAD = 10240
_EPAD = 163840
_CH = 128
_TILES = 16
_CPT = _EPAD // (_TILES * _CH)
_ZC = _NPAD // (_TILES * _CH)
_RB = 1024
_RG = _NPAD // _RB

_mesh = plsc.VectorSubcoreMesh(core_axis_name="c", subcore_axis_name="s")


@functools.partial(
    pl.kernel, mesh=_mesh,
    out_type=jax.ShapeDtypeStruct((2, _NPAD, 128), jnp.float32),
    scratch_types=[
        pltpu.VMEM((_CPT // 2, _CH), jnp.int32),
        pltpu.VMEM((_CH, 128), jnp.float32),
        pltpu.VMEM((_CH, 128), jnp.float32),
        pltpu.VMEM_SHARED((_NPAD, 128), jnp.float32),
    ])
def _deg_kernel(didx_hbm, consts_hbm, deg_hbm, idx_v, zbuf, obuf, acc):
    cid = lax.axis_index("c")
    sid = lax.axis_index("s")
    pltpu.sync_copy(consts_hbm.at[0], zbuf)
    pltpu.sync_copy(consts_hbm.at[1], obuf)
    w = cid * _TILES + sid
    pltpu.sync_copy(didx_hbm.at[pl.ds(w * (_CPT // 2), _CPT // 2)], idx_v)
    for z in range(_ZC):
        pltpu.sync_copy(zbuf, acc.at[pl.ds((sid * _ZC + z) * _CH, _CH)])
    plsc.subcore_barrier()

    @pl.loop(0, _CPT // 2)
    def _(j):
        pltpu.sync_copy(obuf, acc.at[idx_v.at[j]], add=True)

    plsc.subcore_barrier()
    for z in range(_ZC):
        r = pl.ds((sid * _ZC + z) * _CH, _CH)
        pltpu.sync_copy(acc.at[r], zbuf)
        pltpu.sync_copy(zbuf, deg_hbm.at[cid].at[r])


def _make_prop(nslices):
    per_core = nslices // 2

    @functools.partial(
        pl.kernel, mesh=_mesh,
        out_type=jax.ShapeDtypeStruct((nslices, _NPAD, 128), jnp.float32),
        scratch_types=[
            pltpu.VMEM((_CPT, _CH), jnp.int32),
            pltpu.VMEM((_CPT, _CH), jnp.int32),
            pltpu.VMEM((_CH, 128), jnp.float32),
            pltpu.VMEM_SHARED((_NPAD, 128), jnp.float32),
        ])
    def _prop(g_hbm, sidx_hbm, didx_hbm, z_hbm, out_hbm,
              sidx_v, didx_v, buf, acc):
        cid = lax.axis_index("c")
        sid = lax.axis_index("s")
        pltpu.sync_copy(sidx_hbm.at[sid], sidx_v)
        pltpu.sync_copy(didx_hbm.at[sid], didx_v)
        for p in range(per_core):
            k = cid * per_core + p
            gk = g_hbm.at[k]
            pltpu.sync_copy(z_hbm, buf)
            for z in range(_ZC):
                pltpu.sync_copy(buf, acc.at[pl.ds((sid * _ZC + z) * _CH, _CH)])
            plsc.subcore_barrier()

            @pl.loop(0, _CPT)
            def _(j):
                pltpu.sync_copy(gk.at[sidx_v.at[j]], buf)
                pltpu.sync_copy(buf, acc.at[didx_v.at[j]], add=True)

            plsc.subcore_barrier()
            for z in range(_ZC):
                r = pl.ds((sid * _ZC + z) * _CH, _CH)
                pltpu.sync_copy(acc.at[r], buf)
                pltpu.sync_copy(buf, out_hbm.at[k].at[r])
            plsc.subcore_barrier()

    return _prop


_prop4 = _make_prop(4)
_prop2 = _make_prop(2)


def _dis(deg_ref):
    return lax.rsqrt(deg_ref[0, :, 0:1] + deg_ref[1, :, 0:1] + 1.0)


def _tc_in(x_pad, W1, deg):
    def body(x_ref, w_ref, deg_ref, o_ref):
        dis = _dis(deg_ref)
        o_ref[0] = dis * jnp.dot(x_ref[...], w_ref[...],
                                 preferred_element_type=jnp.float32,
                                 precision=lax.Precision.HIGHEST)

    return pl.pallas_call(
        body,
        grid=(4, _RG),
        in_specs=[
            pl.BlockSpec((_RB, 256), lambda k, r: (r, 0)),
            pl.BlockSpec((256, 128), lambda k, r: (0, k)),
            pl.BlockSpec((2, _RB, 128), lambda k, r: (0, r, 0)),
        ],
        out_specs=pl.BlockSpec((1, _RB, 128), lambda k, r: (k, r, 0)),
        out_shape=jax.ShapeDtypeStruct((4, _NPAD, 128), jnp.float32),
    )(x_pad, W1, deg)


def _tc_mid(P, g, deg, b, W, kin_n, kout_n, relu):
    def body(p_ref, g_ref, deg_ref, b_ref, w_ref, o_ref):
        dis = _dis(deg_ref)
        W_all = w_ref[...]
        b_all = b_ref[...]
        acc = jnp.zeros((_RB, 128), jnp.float32)
        for kin in range(kin_n):
            t = dis * (p_ref[kin] + g_ref[kin]) + b_all[:, kin * 128:(kin + 1) * 128]
            if relu:
                t = jnp.maximum(t, 0.0)
            acc = acc + jnp.dot(t, W_all[kin * 128:(kin + 1) * 128, :],
                                preferred_element_type=jnp.float32,
                                precision=lax.Precision.HIGHEST)
        o_ref[0] = dis * acc

    return pl.pallas_call(
        body,
        grid=(kout_n, _RG),
        in_specs=[
            pl.BlockSpec((kin_n, _RB, 128), lambda k, r: (0, r, 0)),
            pl.BlockSpec((kin_n, _RB, 128), lambda k, r: (0, r, 0)),
            pl.BlockSpec((2, _RB, 128), lambda k, r: (0, r, 0)),
            pl.BlockSpec((1, kin_n * 128), lambda k, r: (0, 0)),
            pl.BlockSpec((kin_n * 128, 128), lambda k, r: (0, k)),
        ],
        out_specs=pl.BlockSpec((1, _RB, 128), lambda k, r: (k, r, 0)),
        out_shape=jax.ShapeDtypeStruct((kout_n, _NPAD, 128), jnp.float32),
    )(P, g, deg, b, W)


def _tc_out(P, g, deg, b):
    def body(p_ref, g_ref, deg_ref, b_ref, o_ref):
        dis = _dis(deg_ref)
        o_ref[...] = jax.nn.sigmoid(dis * (p_ref[0] + g_ref[0]) + b_ref[...])

    return pl.pallas_call(
        body,
        grid=(2, _RG),
        in_specs=[
            pl.BlockSpec((1, _RB, 128), lambda k, r: (k, r, 0)),
            pl.BlockSpec((1, _RB, 128), lambda k, r: (k, r, 0)),
            pl.BlockSpec((2, _RB, 128), lambda k, r: (0, r, 0)),
            pl.BlockSpec((1, 128), lambda k, r: (0, k)),
        ],
        out_specs=pl.BlockSpec((_RB, 128), lambda k, r: (r, k)),
        out_shape=jax.ShapeDtypeStruct((_NPAD, 256), jnp.float32),
    )(P, g, deg, b)


def kernel(x, edge_index, W1, b1, W2, b2, W3, b3):
    s = edge_index[0]
    d = edge_index[1]
    pad = _EPAD - _E
    s_pad = jnp.concatenate([s, jnp.zeros((pad,), jnp.int32)])
    d_pad = jnp.concatenate([d, jnp.full((pad,), _N, jnp.int32)])
    sidx = s_pad.reshape(_TILES, _CPT, _CH)
    didx = d_pad.reshape(_TILES, _CPT, _CH)
    didx_deg = d_pad.reshape(_EPAD // _CH, _CH)
    z128 = jnp.zeros((_CH, 128), jnp.float32)
    consts = jnp.stack([z128, jnp.ones((_CH, 128), jnp.float32)])

    deg = _deg_kernel(didx_deg, consts)
    x_pad = jnp.pad(x, ((0, _NPAD - _N), (0, 0)))
    g1 = _tc_in(x_pad, W1, deg)
    P1 = _prop4(g1, sidx, didx, z128)
    g2 = _tc_mid(P1, g1, deg, b1.reshape(1, -1), W2, 4, 4, True)
    P2 = _prop4(g2, sidx, didx, z128)
    g3 = _tc_mid(P2, g2, deg, b2.reshape(1, -1), W3, 4, 2, False)
    P3 = _prop2(g3, sidx, didx, z128)
    out = _tc_out(P3, g3, deg, b3.reshape(1, -1))
    return out[:_N]


# speedup vs baseline: 5.3608x; 1.1027x over previous
import functools

import jax
import jax.numpy as jnp
from jax import lax
from jax.experimental import pallas as pl
from jax.experimental.pallas import tpu as pltpu
from jax.experimental.pallas import tpu_sc as plsc

_N = 10000
_E = 160000
_NPAD = 10240
_EPAD = 163840
_CH = 128
_TILES = 16
_CPT = _EPAD // (_TILES * _CH)
_ZC = _NPAD // (_TILES * _CH)
_RB = 1024
_RG = _NPAD // _RB

_mesh = plsc.VectorSubcoreMesh(core_axis_name="c", subcore_axis_name="s")


@functools.partial(
    pl.kernel, mesh=_mesh,
    out_type=jax.ShapeDtypeStruct((2, _NPAD, 128), jnp.float32),
    scratch_types=[
        pltpu.VMEM((_CPT // 2, _CH), jnp.int32),
        pltpu.VMEM((_CH, 128), jnp.float32),
        pltpu.VMEM((_CH, 128), jnp.float32),
        pltpu.VMEM_SHARED((_NPAD, 128), jnp.float32),
    ])
def _deg_kernel(didx_hbm, consts_hbm, deg_hbm, idx_v, zbuf, obuf, acc):
    cid = lax.axis_index("c")
    sid = lax.axis_index("s")
    pltpu.sync_copy(consts_hbm.at[0], zbuf)
    pltpu.sync_copy(consts_hbm.at[1], obuf)
    w = cid * _TILES + sid
    pltpu.sync_copy(didx_hbm.at[pl.ds(w * (_CPT // 2), _CPT // 2)], idx_v)
    for z in range(_ZC):
        pltpu.sync_copy(zbuf, acc.at[pl.ds((sid * _ZC + z) * _CH, _CH)])
    plsc.subcore_barrier()

    @pl.loop(0, _CPT // 2)
    def _(j):
        pltpu.sync_copy(obuf, acc.at[idx_v.at[j]], add=True)

    plsc.subcore_barrier()
    for z in range(_ZC):
        r = pl.ds((sid * _ZC + z) * _CH, _CH)
        pltpu.sync_copy(acc.at[r], zbuf)
        pltpu.sync_copy(zbuf, deg_hbm.at[cid].at[r])


def _make_prop(nslices):
    per_core = nslices // 2

    @functools.partial(
        pl.kernel, mesh=_mesh,
        out_type=jax.ShapeDtypeStruct((nslices, _NPAD, 128), jnp.float32),
        scratch_types=[
            pltpu.VMEM((_CPT, _CH), jnp.int32),
            pltpu.VMEM((2, _CH), jnp.int32),
            pltpu.VMEM((_CH, 128), jnp.float32),
            pltpu.VMEM((_CH, 128), jnp.float32),
            pltpu.VMEM_SHARED((_NPAD, 128), jnp.float32),
            pltpu.SemaphoreType.DMA,
            pltpu.SemaphoreType.DMA,
            pltpu.SemaphoreType.DMA,
            pltpu.SemaphoreType.DMA,
            pltpu.SemaphoreType.DMA,
            pltpu.SemaphoreType.DMA,
        ])
    def _prop(g_hbm, sidx_hbm, didx_hbm, z_hbm, out_hbm,
              sidx_v, didxb, b0, b1, acc, g0s, g1s, d0s, d1s, s0s, s1s):
        cid = lax.axis_index("c")
        sid = lax.axis_index("s")
        di = didx_hbm.at[sid]
        pltpu.sync_copy(sidx_hbm.at[sid], sidx_v)
        for p in range(per_core):
            k = cid * per_core + p
            gk = g_hbm.at[k]
            pltpu.sync_copy(z_hbm, b0)
            for z in range(_ZC):
                pltpu.sync_copy(b0, acc.at[pl.ds((sid * _ZC + z) * _CH, _CH)])
            plsc.subcore_barrier()

            pltpu.async_copy(di.at[0], didxb.at[0], d0s)
            pltpu.async_copy(di.at[1], didxb.at[1], d1s)
            pltpu.async_copy(gk.at[sidx_v.at[0]], b0, g0s)
            pltpu.async_copy(gk.at[sidx_v.at[1]], b1, g1s)

            @pl.loop(0, _CPT, step=2)
            def _(j):
                pltpu.make_async_copy(gk.at[sidx_v.at[j]], b0, g0s).wait()
                pltpu.make_async_copy(di.at[j], didxb.at[0], d0s).wait()
                pltpu.async_copy(b0, acc.at[didxb.at[0]], s0s, add=True)
                pltpu.make_async_copy(gk.at[sidx_v.at[j + 1]], b1, g1s).wait()
                pltpu.make_async_copy(di.at[j + 1], didxb.at[1], d1s).wait()
                pltpu.async_copy(b1, acc.at[didxb.at[1]], s1s, add=True)
                pltpu.make_async_copy(b0, acc.at[didxb.at[0]], s0s).wait()

                @pl.when(j + 2 < _CPT)
                def _():
                    pltpu.async_copy(di.at[j + 2], didxb.at[0], d0s)
                    pltpu.async_copy(gk.at[sidx_v.at[j + 2]], b0, g0s)

                pltpu.make_async_copy(b1, acc.at[didxb.at[1]], s1s).wait()

                @pl.when(j + 3 < _CPT)
                def _():
                    pltpu.async_copy(di.at[j + 3], didxb.at[1], d1s)
                    pltpu.async_copy(gk.at[sidx_v.at[j + 3]], b1, g1s)

            plsc.subcore_barrier()
            for z in range(_ZC):
                r = pl.ds((sid * _ZC + z) * _CH, _CH)
                pltpu.sync_copy(acc.at[r], b0)
                pltpu.sync_copy(b0, out_hbm.at[k].at[r])
            plsc.subcore_barrier()

    return _prop


_prop4 = _make_prop(4)
_prop2 = _make_prop(2)


def _dis(deg_ref):
    return lax.rsqrt(deg_ref[0, :, 0:1] + deg_ref[1, :, 0:1] + 1.0)


def _tc_in(x_pad, W1, deg):
    def body(x_ref, w_ref, deg_ref, o_ref):
        dis = _dis(deg_ref)
        o_ref[0] = dis * jnp.dot(x_ref[...], w_ref[...],
                                 preferred_element_type=jnp.float32,
                                 precision=lax.Precision.HIGHEST)

    return pl.pallas_call(
        body,
        grid=(4, _RG),
        in_specs=[
            pl.BlockSpec((_RB, 256), lambda k, r: (r, 0)),
            pl.BlockSpec((256, 128), lambda k, r: (0, k)),
            pl.BlockSpec((2, _RB, 128), lambda k, r: (0, r, 0)),
        ],
        out_specs=pl.BlockSpec((1, _RB, 128), lambda k, r: (k, r, 0)),
        out_shape=jax.ShapeDtypeStruct((4, _NPAD, 128), jnp.float32),
    )(x_pad, W1, deg)


def _tc_mid(P, g, deg, b, W, kin_n, kout_n, relu):
    def body(p_ref, g_ref, deg_ref, b_ref, w_ref, o_ref):
        dis = _dis(deg_ref)
        W_all = w_ref[...]
        b_all = b_ref[...]
        acc = jnp.zeros((_RB, 128), jnp.float32)
        for kin in range(kin_n):
            t = dis * (p_ref[kin] + g_ref[kin]) + b_all[:, kin * 128:(kin + 1) * 128]
            if relu:
                t = jnp.maximum(t, 0.0)
            acc = acc + jnp.dot(t, W_all[kin * 128:(kin + 1) * 128, :],
                                preferred_element_type=jnp.float32,
                                precision=lax.Precision.HIGHEST)
        o_ref[0] = dis * acc

    return pl.pallas_call(
        body,
        grid=(kout_n, _RG),
        in_specs=[
            pl.BlockSpec((kin_n, _RB, 128), lambda k, r: (0, r, 0)),
            pl.BlockSpec((kin_n, _RB, 128), lambda k, r: (0, r, 0)),
            pl.BlockSpec((2, _RB, 128), lambda k, r: (0, r, 0)),
            pl.BlockSpec((1, kin_n * 128), lambda k, r: (0, 0)),
            pl.BlockSpec((kin_n * 128, 128), lambda k, r: (0, k)),
        ],
        out_specs=pl.BlockSpec((1, _RB, 128), lambda k, r: (k, r, 0)),
        out_shape=jax.ShapeDtypeStruct((kout_n, _NPAD, 128), jnp.float32),
    )(P, g, deg, b, W)


def _tc_out(P, g, deg, b):
    def body(p_ref, g_ref, deg_ref, b_ref, o_ref):
        dis = _dis(deg_ref)
        o_ref[...] = jax.nn.sigmoid(dis * (p_ref[0] + g_ref[0]) + b_ref[...])

    return pl.pallas_call(
        body,
        grid=(2, _RG),
        in_specs=[
            pl.BlockSpec((1, _RB, 128), lambda k, r: (k, r, 0)),
            pl.BlockSpec((1, _RB, 128), lambda k, r: (k, r, 0)),
            pl.BlockSpec((2, _RB, 128), lambda k, r: (0, r, 0)),
            pl.BlockSpec((1, 128), lambda k, r: (0, k)),
        ],
        out_specs=pl.BlockSpec((_RB, 128), lambda k, r: (r, k)),
        out_shape=jax.ShapeDtypeStruct((_NPAD, 256), jnp.float32),
    )(P, g, deg, b)


def kernel(x, edge_index, W1, b1, W2, b2, W3, b3):
    s = edge_index[0]
    d = edge_index[1]
    pad = _EPAD - _E
    s_pad = jnp.concatenate([s, jnp.zeros((pad,), jnp.int32)])
    d_pad = jnp.concatenate([d, jnp.full((pad,), _N, jnp.int32)])
    sidx = s_pad.reshape(_TILES, _CPT, _CH)
    didx = d_pad.reshape(_TILES, _CPT, _CH)
    didx_deg = d_pad.reshape(_EPAD // _CH, _CH)
    z128 = jnp.zeros((_CH, 128), jnp.float32)
    consts = jnp.stack([z128, jnp.ones((_CH, 128), jnp.float32)])

    deg = _deg_kernel(didx_deg, consts)
    x_pad = jnp.pad(x, ((0, _NPAD - _N), (0, 0)))
    g1 = _tc_in(x_pad, W1, deg)
    P1 = _prop4(g1, sidx, didx, z128)
    g2 = _tc_mid(P1, g1, deg, b1.reshape(1, -1), W2, 4, 4, True)
    P2 = _prop4(g2, sidx, didx, z128)
    g3 = _tc_mid(P2, g2, deg, b2.reshape(1, -1), W3, 4, 2, False)
    P3 = _prop2(g3, sidx, didx, z128)
    out = _tc_out(P3, g3, deg, b3.reshape(1, -1))
    return out[:_N]


# speedup vs baseline: 6.7780x; 1.2644x over previous
import functools

import jax
import jax.numpy as jnp
from jax import lax
from jax.experimental import pallas as pl
from jax.experimental.pallas import tpu as pltpu
from jax.experimental.pallas import tpu_sc as plsc

_N = 10000
_E = 160000
_NPAD = 10240
_EPAD = 163840
_CH = 128
_DEPTH = 4
_TILES = 16
_CPT = _EPAD // (_TILES * _CH)
_RPT = _NPAD // _TILES
_ZC = _RPT // _CH
_RB = 1024
_RG = _NPAD // _RB

_mesh = plsc.VectorSubcoreMesh(core_axis_name="c", subcore_axis_name="s")


@functools.partial(
    pl.kernel, mesh=_mesh,
    out_type=jax.ShapeDtypeStruct((2, _NPAD, 128), jnp.float32),
    scratch_types=[
        pltpu.VMEM((_CPT // 2, _CH), jnp.int32),
        pltpu.VMEM((_CH, 128), jnp.float32),
        pltpu.VMEM((_CH, 128), jnp.float32),
        pltpu.VMEM_SHARED((_NPAD, 128), jnp.float32),
    ])
def _deg_kernel(didx_hbm, consts_hbm, deg_hbm, idx_v, zbuf, obuf, acc):
    cid = lax.axis_index("c")
    sid = lax.axis_index("s")
    pltpu.sync_copy(consts_hbm.at[0], zbuf)
    pltpu.sync_copy(consts_hbm.at[1], obuf)
    w = cid * _TILES + sid
    pltpu.sync_copy(didx_hbm.at[pl.ds(w * (_CPT // 2), _CPT // 2)], idx_v)
    for z in range(_ZC):
        pltpu.sync_copy(zbuf, acc.at[pl.ds((sid * _ZC + z) * _CH, _CH)])
    plsc.subcore_barrier()

    @pl.loop(0, _CPT // 2)
    def _(j):
        pltpu.sync_copy(obuf, acc.at[idx_v.at[j]], add=True)

    plsc.subcore_barrier()
    for z in range(_ZC):
        r = pl.ds((sid * _ZC + z) * _CH, _CH)
        pltpu.sync_copy(acc.at[r], zbuf)
        pltpu.sync_copy(zbuf, deg_hbm.at[cid].at[r])


def _make_prop(nslices):
    nhalf = nslices * 2
    per_core = nhalf // 2

    @functools.partial(
        pl.kernel, mesh=_mesh,
        compiler_params=pltpu.CompilerParams(use_tc_tiling_on_sc=False),
        out_type=jax.ShapeDtypeStruct((nhalf, _NPAD, 64), jnp.float32),
        scratch_types=[
            pltpu.VMEM((_CPT, _CH), jnp.int32),
            pltpu.VMEM((_DEPTH, _CH), jnp.int32),
        ] + [pltpu.VMEM((_CH, 64), jnp.float32) for _ in range(_DEPTH)] + [
            pltpu.VMEM_SHARED((_NPAD, 64), jnp.float32),
            pltpu.VMEM_SHARED((_NPAD, 64), jnp.float32),
        ] + [pltpu.SemaphoreType.DMA for _ in range(3 * _DEPTH)])
    def _prop(g_hbm, sidx_hbm, didx_hbm, z_hbm, out_hbm,
              sidx_v, didxb, *rest):
        bufs = rest[:_DEPTH]
        gsp = rest[_DEPTH]
        acc = rest[_DEPTH + 1]
        gs = rest[_DEPTH + 2:_DEPTH + 2 + _DEPTH]
        ds = rest[_DEPTH + 2 + _DEPTH:_DEPTH + 2 + 2 * _DEPTH]
        ss = rest[_DEPTH + 2 + 2 * _DEPTH:]
        cid = lax.axis_index("c")
        sid = lax.axis_index("s")
        di = didx_hbm.at[sid]
        pltpu.sync_copy(sidx_hbm.at[sid], sidx_v)
        rows0 = sid * _RPT
        for p in range(per_core):
            hs = cid * per_core + p
            for z in range(_ZC):
                r = pl.ds(rows0 + z * _CH, _CH)
                pltpu.sync_copy(g_hbm.at[hs].at[r], bufs[1])
                pltpu.sync_copy(bufs[1], gsp.at[r])
            pltpu.sync_copy(z_hbm, bufs[0])
            for z in range(_ZC):
                pltpu.sync_copy(bufs[0],
                                acc.at[pl.ds(rows0 + z * _CH, _CH)])
            plsc.subcore_barrier()

            for t in range(_DEPTH):
                pltpu.async_copy(di.at[t], didxb.at[t], ds[t])
                pltpu.async_copy(gsp.at[sidx_v.at[t]], bufs[t], gs[t])

            @pl.loop(0, _CPT, step=_DEPTH)
            def _(j):
                for t in range(_DEPTH):
                    pltpu.make_async_copy(gsp.at[sidx_v.at[j + t]],
                                          bufs[t], gs[t]).wait()
                    pltpu.make_async_copy(di.at[j + t], didxb.at[t],
                                          ds[t]).wait()
                    pltpu.async_copy(bufs[t], acc.at[didxb.at[t]],
                                     ss[t], add=True)
                for t in range(_DEPTH):
                    pltpu.make_async_copy(bufs[t], acc.at[didxb.at[t]],
                                          ss[t]).wait()

                    @pl.when(j + _DEPTH + t < _CPT)
                    def _(t=t):
                        pltpu.async_copy(di.at[j + _DEPTH + t],
                                         didxb.at[t], ds[t])
                        pltpu.async_copy(gsp.at[sidx_v.at[j + _DEPTH + t]],
                                         bufs[t], gs[t])

            plsc.subcore_barrier()
            for z in range(_ZC):
                r = pl.ds(rows0 + z * _CH, _CH)
                pltpu.sync_copy(acc.at[r], bufs[0])
                pltpu.sync_copy(bufs[0], out_hbm.at[hs].at[r])
            plsc.subcore_barrier()

    return _prop


_prop4 = _make_prop(4)
_prop2 = _make_prop(2)


def _dis(deg_ref):
    return lax.rsqrt(deg_ref[0, :, 0:1] + deg_ref[1, :, 0:1] + 1.0)


def _tc_in(x_pad, W1, deg):
    def body(x_ref, w_ref, deg_ref, o_ref):
        dis = _dis(deg_ref)
        res = dis * jnp.dot(x_ref[...], w_ref[...],
                            preferred_element_type=jnp.float32,
                            precision=lax.Precision.HIGHEST)
        o_ref[0] = res[:, :64]
        o_ref[1] = res[:, 64:]

    return pl.pallas_call(
        body,
        grid=(4, _RG),
        in_specs=[
            pl.BlockSpec((_RB, 256), lambda k, r: (r, 0)),
            pl.BlockSpec((256, 128), lambda k, r: (0, k)),
            pl.BlockSpec((2, _RB, 128), lambda k, r: (0, r, 0)),
        ],
        out_specs=pl.BlockSpec((2, _RB, 64), lambda k, r: (k, r, 0)),
        out_shape=jax.ShapeDtypeStruct((8, _NPAD, 64), jnp.float32),
    )(x_pad, W1, deg)


def _tc_mid(P, g, deg, b, W, kin_n, kout_n, relu):
    def body(p_ref, g_ref, deg_ref, b_ref, w_ref, o_ref):
        dis = _dis(deg_ref)
        W_all = w_ref[...]
        b_all = b_ref[...]
        acc = jnp.zeros((_RB, 128), jnp.float32)
        for q in range(2 * kin_n):
            t = dis * (p_ref[q] + g_ref[q]) + b_all[:, q * 64:(q + 1) * 64]
            if relu:
                t = jnp.maximum(t, 0.0)
            acc = acc + jnp.dot(t, W_all[q * 64:(q + 1) * 64, :],
                                preferred_element_type=jnp.float32,
                                precision=lax.Precision.HIGHEST)
        res = dis * acc
        o_ref[0] = res[:, :64]
        o_ref[1] = res[:, 64:]

    return pl.pallas_call(
        body,
        grid=(kout_n, _RG),
        in_specs=[
            pl.BlockSpec((2 * kin_n, _RB, 64), lambda k, r: (0, r, 0)),
            pl.BlockSpec((2 * kin_n, _RB, 64), lambda k, r: (0, r, 0)),
            pl.BlockSpec((2, _RB, 128), lambda k, r: (0, r, 0)),
            pl.BlockSpec((1, kin_n * 128), lambda k, r: (0, 0)),
            pl.BlockSpec((kin_n * 128, 128), lambda k, r: (0, k)),
        ],
        out_specs=pl.BlockSpec((2, _RB, 64), lambda k, r: (k, r, 0)),
        out_shape=jax.ShapeDtypeStruct((2 * kout_n, _NPAD, 64), jnp.float32),
    )(P, g, deg, b, W)


def _tc_out(P, g, deg, b):
    def body(p_ref, g_ref, deg_ref, b_ref, o_ref):
        dis = _dis(deg_ref)
        t = jnp.concatenate(
            [p_ref[0] + g_ref[0], p_ref[1] + g_ref[1]], axis=1)
        o_ref[...] = jax.nn.sigmoid(dis * t + b_ref[...])

    return pl.pallas_call(
        body,
        grid=(2, _RG),
        in_specs=[
            pl.BlockSpec((2, _RB, 64), lambda k, r: (k, r, 0)),
            pl.BlockSpec((2, _RB, 64), lambda k, r: (k, r, 0)),
            pl.BlockSpec((2, _RB, 128), lambda k, r: (0, r, 0)),
            pl.BlockSpec((1, 128), lambda k, r: (0, k)),
        ],
        out_specs=pl.BlockSpec((_RB, 128), lambda k, r: (r, k)),
        out_shape=jax.ShapeDtypeStruct((_NPAD, 256), jnp.float32),
    )(P, g, deg, b)


def kernel(x, edge_index, W1, b1, W2, b2, W3, b3):
    s = edge_index[0]
    d = edge_index[1]
    pad = _EPAD - _E
    s_pad = jnp.concatenate([s, jnp.zeros((pad,), jnp.int32)])
    d_pad = jnp.concatenate([d, jnp.full((pad,), _N, jnp.int32)])
    sidx = s_pad.reshape(_TILES, _CPT, _CH)
    didx = d_pad.reshape(_TILES, _CPT, _CH)
    didx_deg = d_pad.reshape(_EPAD // _CH, _CH)
    z128 = jnp.zeros((_CH, 64), jnp.float32)
    consts = jnp.stack([jnp.zeros((_CH, 128), jnp.float32),
                        jnp.ones((_CH, 128), jnp.float32)])

    deg = _deg_kernel(didx_deg, consts)
    x_pad = jnp.pad(x, ((0, _NPAD - _N), (0, 0)))
    g1 = _tc_in(x_pad, W1, deg)
    P1 = _prop4(g1, sidx, didx, z128)
    g2 = _tc_mid(P1, g1, deg, b1.reshape(1, -1), W2, 4, 4, True)
    P2 = _prop4(g2, sidx, didx, z128)
    g3 = _tc_mid(P2, g2, deg, b2.reshape(1, -1), W3, 4, 2, False)
    P3 = _prop2(g3, sidx, didx, z128)
    out = _tc_out(P3, g3, deg, b3.reshape(1, -1))
    return out[:_N]


# speedup vs baseline: 8.2451x; 1.2164x over previous
import functools

import jax
import jax.numpy as jnp
from jax import lax
from jax.experimental import pallas as pl
from jax.experimental.pallas import tpu as pltpu
from jax.experimental.pallas import tpu_sc as plsc

_N = 10000
_E = 160000
_NPAD = 10240
_EPAD = 163840
_CH = 128
_DEPTH = 4
_TILES = 16
_CPT = _EPAD // (_TILES * _CH)
_RPT = _NPAD // _TILES
_ZC = _RPT // _CH
_RB = 1024
_RG = _NPAD // _RB

_mesh = plsc.VectorSubcoreMesh(core_axis_name="c", subcore_axis_name="s")


@functools.partial(
    pl.kernel, mesh=_mesh,
    out_type=jax.ShapeDtypeStruct((2, _NPAD, 128), jnp.float32),
    scratch_types=[
        pltpu.VMEM((_CPT // 2, _CH), jnp.int32),
        pltpu.VMEM((_CH, 128), jnp.float32),
        pltpu.VMEM((_CH, 128), jnp.float32),
        pltpu.VMEM_SHARED((_NPAD, 128), jnp.float32),
    ])
def _deg_kernel(didx_hbm, consts_hbm, deg_hbm, idx_v, zbuf, obuf, acc):
    cid = lax.axis_index("c")
    sid = lax.axis_index("s")
    pltpu.sync_copy(consts_hbm.at[0], zbuf)
    pltpu.sync_copy(consts_hbm.at[1], obuf)
    w = cid * _TILES + sid
    pltpu.sync_copy(didx_hbm.at[pl.ds(w * (_CPT // 2), _CPT // 2)], idx_v)
    for z in range(_ZC):
        pltpu.sync_copy(zbuf, acc.at[pl.ds((sid * _ZC + z) * _CH, _CH)])
    plsc.subcore_barrier()

    @pl.loop(0, _CPT // 2)
    def _(j):
        pltpu.sync_copy(obuf, acc.at[idx_v.at[j]], add=True)

    plsc.subcore_barrier()
    for z in range(_ZC):
        r = pl.ds((sid * _ZC + z) * _CH, _CH)
        pltpu.sync_copy(acc.at[r], zbuf)
        pltpu.sync_copy(zbuf, deg_hbm.at[cid].at[r])


def _make_prop(nslices):
    nhalf = nslices * 2
    per_core = nhalf // 2

    @functools.partial(
        pl.kernel, mesh=_mesh,
        compiler_params=pltpu.CompilerParams(use_tc_tiling_on_sc=False),
        out_type=jax.ShapeDtypeStruct((nslices, _NPAD, 128), jnp.float32),
        scratch_types=[
            pltpu.VMEM((_CPT, _CH), jnp.int32),
            pltpu.VMEM((_DEPTH, _CH), jnp.int32),
        ] + [pltpu.VMEM((_CH, 64), jnp.float32) for _ in range(_DEPTH)] + [
            pltpu.VMEM_SHARED((_NPAD, 64), jnp.float32),
            pltpu.VMEM_SHARED((_NPAD, 64), jnp.float32),
        ] + [pltpu.SemaphoreType.DMA for _ in range(3 * _DEPTH)])
    def _prop(g_hbm, sidx_hbm, didx_hbm, z_hbm, out_hbm,
              sidx_v, didxb, *rest):
        bufs = rest[:_DEPTH]
        gsp = rest[_DEPTH]
        acc = rest[_DEPTH + 1]
        gs = rest[_DEPTH + 2:_DEPTH + 2 + _DEPTH]
        ds = rest[_DEPTH + 2 + _DEPTH:_DEPTH + 2 + 2 * _DEPTH]
        ss = rest[_DEPTH + 2 + 2 * _DEPTH:]
        cid = lax.axis_index("c")
        sid = lax.axis_index("s")
        di = didx_hbm.at[sid]
        pltpu.sync_copy(sidx_hbm.at[sid], sidx_v)
        rows0 = sid * _RPT
        for p in range(per_core):
            hs = cid * per_core + p
            kk = hs // 2
            hl = pl.ds((hs % 2) * 64, 64)
            for z in range(_ZC):
                r = pl.ds(rows0 + z * _CH, _CH)
                pltpu.sync_copy(g_hbm.at[kk].at[r, hl], bufs[1])
                pltpu.sync_copy(bufs[1], gsp.at[r])
            pltpu.sync_copy(z_hbm, bufs[0])
            for z in range(_ZC):
                pltpu.sync_copy(bufs[0],
                                acc.at[pl.ds(rows0 + z * _CH, _CH)])
            plsc.subcore_barrier()

            for t in range(_DEPTH):
                pltpu.async_copy(di.at[t], didxb.at[t], ds[t])
                pltpu.async_copy(gsp.at[sidx_v.at[t]], bufs[t], gs[t])

            @pl.loop(0, _CPT, step=_DEPTH)
            def _(j):
                for t in range(_DEPTH):
                    pltpu.make_async_copy(gsp.at[sidx_v.at[j + t]],
                                          bufs[t], gs[t]).wait()
                    pltpu.make_async_copy(di.at[j + t], didxb.at[t],
                                          ds[t]).wait()
                    pltpu.async_copy(bufs[t], acc.at[didxb.at[t]],
                                     ss[t], add=True)
                for t in range(_DEPTH):
                    pltpu.make_async_copy(bufs[t], acc.at[didxb.at[t]],
                                          ss[t]).wait()

                    @pl.when(j + _DEPTH + t < _CPT)
                    def _(t=t):
                        pltpu.async_copy(di.at[j + _DEPTH + t],
                                         didxb.at[t], ds[t])
                        pltpu.async_copy(gsp.at[sidx_v.at[j + _DEPTH + t]],
                                         bufs[t], gs[t])

            plsc.subcore_barrier()
            for z in range(_ZC):
                r = pl.ds(rows0 + z * _CH, _CH)
                pltpu.sync_copy(acc.at[r], bufs[0])
                pltpu.sync_copy(bufs[0], out_hbm.at[kk].at[r, hl])
            plsc.subcore_barrier()

    return _prop


_prop4 = _make_prop(4)
_prop2 = _make_prop(2)


def _dis(deg_ref):
    return lax.rsqrt(deg_ref[0, :, 0:1] + deg_ref[1, :, 0:1] + 1.0)


def _tc_in(x_pad, W1, deg):
    def body(x_ref, w_ref, deg_ref, o_ref):
        dis = _dis(deg_ref)
        o_ref[0] = dis * jnp.dot(x_ref[...], w_ref[...],
                                 preferred_element_type=jnp.float32,
                                 precision=lax.Precision.HIGHEST)

    return pl.pallas_call(
        body,
        grid=(4, _RG),
        in_specs=[
            pl.BlockSpec((_RB, 256), lambda k, r: (r, 0)),
            pl.BlockSpec((256, 128), lambda k, r: (0, k)),
            pl.BlockSpec((2, _RB, 128), lambda k, r: (0, r, 0)),
        ],
        out_specs=pl.BlockSpec((1, _RB, 128), lambda k, r: (k, r, 0)),
        out_shape=jax.ShapeDtypeStruct((4, _NPAD, 128), jnp.float32),
    )(x_pad, W1, deg)


def _tc_mid(P, g, deg, b, W, kin_n, kout_n, relu):
    def body(p_ref, g_ref, deg_ref, b_ref, w_ref, o_ref):
        dis = _dis(deg_ref)
        W_all = w_ref[...]
        b_all = b_ref[...]
        acc = jnp.zeros((_RB, 128), jnp.float32)
        for kin in range(kin_n):
            t = dis * (p_ref[kin] + g_ref[kin]) + b_all[:, kin * 128:(kin + 1) * 128]
            if relu:
                t = jnp.maximum(t, 0.0)
            acc = acc + jnp.dot(t, W_all[kin * 128:(kin + 1) * 128, :],
                                preferred_element_type=jnp.float32,
                                precision=lax.Precision.HIGHEST)
        o_ref[0] = dis * acc

    return pl.pallas_call(
        body,
        grid=(kout_n, _RG),
        in_specs=[
            pl.BlockSpec((kin_n, _RB, 128), lambda k, r: (0, r, 0)),
            pl.BlockSpec((kin_n, _RB, 128), lambda k, r: (0, r, 0)),
            pl.BlockSpec((2, _RB, 128), lambda k, r: (0, r, 0)),
            pl.BlockSpec((1, kin_n * 128), lambda k, r: (0, 0)),
            pl.BlockSpec((kin_n * 128, 128), lambda k, r: (0, k)),
        ],
        out_specs=pl.BlockSpec((1, _RB, 128), lambda k, r: (k, r, 0)),
        out_shape=jax.ShapeDtypeStruct((kout_n, _NPAD, 128), jnp.float32),
    )(P, g, deg, b, W)


def _tc_out(P, g, deg, b):
    def body(p_ref, g_ref, deg_ref, b_ref, o_ref):
        dis = _dis(deg_ref)
        o_ref[...] = jax.nn.sigmoid(dis * (p_ref[0] + g_ref[0]) + b_ref[...])

    return pl.pallas_call(
        body,
        grid=(2, _RG),
        in_specs=[
            pl.BlockSpec((1, _RB, 128), lambda k, r: (k, r, 0)),
            pl.BlockSpec((1, _RB, 128), lambda k, r: (k, r, 0)),
            pl.BlockSpec((2, _RB, 128), lambda k, r: (0, r, 0)),
            pl.BlockSpec((1, 128), lambda k, r: (0, k)),
        ],
        out_specs=pl.BlockSpec((_RB, 128), lambda k, r: (r, k)),
        out_shape=jax.ShapeDtypeStruct((_NPAD, 256), jnp.float32),
    )(P, g, deg, b)


def kernel(x, edge_index, W1, b1, W2, b2, W3, b3):
    s = edge_index[0]
    d = edge_index[1]
    pad = _EPAD - _E
    s_pad = jnp.concatenate([s, jnp.zeros((pad,), jnp.int32)])
    d_pad = jnp.concatenate([d, jnp.full((pad,), _N, jnp.int32)])
    sidx = s_pad.reshape(_TILES, _CPT, _CH)
    didx = d_pad.reshape(_TILES, _CPT, _CH)
    didx_deg = d_pad.reshape(_EPAD // _CH, _CH)
    z128 = jnp.zeros((_CH, 64), jnp.float32)
    consts = jnp.stack([jnp.zeros((_CH, 128), jnp.float32),
                        jnp.ones((_CH, 128), jnp.float32)])

    deg = _deg_kernel(didx_deg, consts)
    x_pad = jnp.pad(x, ((0, _NPAD - _N), (0, 0)))
    g1 = _tc_in(x_pad, W1, deg)
    P1 = _prop4(g1, sidx, didx, z128)
    g2 = _tc_mid(P1, g1, deg, b1.reshape(1, -1), W2, 4, 4, True)
    P2 = _prop4(g2, sidx, didx, z128)
    g3 = _tc_mid(P2, g2, deg, b2.reshape(1, -1), W3, 4, 2, False)
    P3 = _prop2(g3, sidx, didx, z128)
    out = _tc_out(P3, g3, deg, b3.reshape(1, -1))
    return out[:_N]


# speedup vs baseline: 8.6532x; 1.0495x over previous
import functools

import jax
import jax.numpy as jnp
from jax import lax
from jax.experimental import pallas as pl
from jax.experimental.pallas import tpu as pltpu
from jax.experimental.pallas import tpu_sc as plsc

_N = 10000
_E = 160000
_NPAD = 10240
_EPAD = 163840
_CH = 128
_DEPTH = 4
_TILES = 16
_CPT = _EPAD // (_TILES * _CH)
_RPT = _NPAD // _TILES
_ZC = _RPT // _CH
_RB = 1024
_RG = _NPAD // _RB

_mesh = plsc.VectorSubcoreMesh(core_axis_name="c", subcore_axis_name="s")


@functools.partial(
    pl.kernel, mesh=_mesh,
    out_type=jax.ShapeDtypeStruct((2, _NPAD, 128), jnp.float32),
    scratch_types=[
        pltpu.VMEM((_CPT // 2, _CH), jnp.int32),
        pltpu.VMEM((_CH, 128), jnp.float32),
        pltpu.VMEM((_CH, 128), jnp.float32),
        pltpu.VMEM_SHARED((_NPAD, 128), jnp.float32),
    ])
def _deg_kernel(didx_hbm, consts_hbm, deg_hbm, idx_v, zbuf, obuf, acc):
    cid = lax.axis_index("c")
    sid = lax.axis_index("s")
    pltpu.sync_copy(consts_hbm.at[0], zbuf)
    pltpu.sync_copy(consts_hbm.at[1], obuf)
    w = cid * _TILES + sid
    pltpu.sync_copy(didx_hbm.at[pl.ds(w * (_CPT // 2), _CPT // 2)], idx_v)
    for z in range(_ZC):
        pltpu.sync_copy(zbuf, acc.at[pl.ds((sid * _ZC + z) * _CH, _CH)])
    plsc.subcore_barrier()

    @pl.loop(0, _CPT // 2)
    def _(j):
        pltpu.sync_copy(obuf, acc.at[idx_v.at[j]], add=True)

    plsc.subcore_barrier()
    for z in range(_ZC):
        r = pl.ds((sid * _ZC + z) * _CH, _CH)
        pltpu.sync_copy(acc.at[r], zbuf)
        pltpu.sync_copy(zbuf, deg_hbm.at[cid].at[r])


def _make_prop(nslices):
    nhalf = nslices * 2
    per_core = nhalf // 2

    @functools.partial(
        pl.kernel, mesh=_mesh,
        compiler_params=pltpu.CompilerParams(use_tc_tiling_on_sc=False),
        out_type=jax.ShapeDtypeStruct((nslices, _NPAD, 128), jnp.float32),
        scratch_types=[
            pltpu.VMEM((_CPT, _CH), jnp.int32),
            pltpu.VMEM((_DEPTH, _CH), jnp.int32),
        ] + [pltpu.VMEM((_CH, 64), jnp.float32) for _ in range(_DEPTH)] + [
            pltpu.VMEM_SHARED((_NPAD, 64), jnp.float32),
            pltpu.VMEM_SHARED((_NPAD, 64), jnp.float32),
        ] + [pltpu.SemaphoreType.DMA for _ in range(3 * _DEPTH)])
    def _prop(g_hbm, sidx_hbm, didx_hbm, z_hbm, out_hbm,
              sidx_v, didxb, *rest):
        bufs = rest[:_DEPTH]
        gsp = rest[_DEPTH]
        acc = rest[_DEPTH + 1]
        gs = rest[_DEPTH + 2:_DEPTH + 2 + _DEPTH]
        ds = rest[_DEPTH + 2 + _DEPTH:_DEPTH + 2 + 2 * _DEPTH]
        ss = rest[_DEPTH + 2 + 2 * _DEPTH:]
        cid = lax.axis_index("c")
        sid = lax.axis_index("s")
        di = didx_hbm.at[sid]
        pltpu.sync_copy(sidx_hbm.at[sid], sidx_v)
        rows0 = sid * _RPT
        for p in range(per_core):
            hs = cid * per_core + p
            kk = hs // 2
            hl = pl.ds((hs % 2) * 64, 64)
            for z in range(_ZC):
                r = pl.ds(rows0 + z * _CH, _CH)
                pltpu.sync_copy(g_hbm.at[kk].at[r, hl], bufs[1])
                pltpu.sync_copy(bufs[1], gsp.at[r])
            pltpu.sync_copy(z_hbm, bufs[0])
            for z in range(_ZC):
                pltpu.sync_copy(bufs[0],
                                acc.at[pl.ds(rows0 + z * _CH, _CH)])
            plsc.subcore_barrier()

            for t in range(_DEPTH):
                pltpu.async_copy(di.at[t], didxb.at[t], ds[t])
                pltpu.async_copy(gsp.at[sidx_v.at[t]], bufs[t], gs[t])

            @pl.loop(0, _CPT, step=_DEPTH)
            def _(j):
                for t in range(_DEPTH):
                    pltpu.make_async_copy(gsp.at[sidx_v.at[j + t]],
                                          bufs[t], gs[t]).wait()
                    pltpu.make_async_copy(di.at[j + t], didxb.at[t],
                                          ds[t]).wait()
                    pltpu.async_copy(bufs[t], acc.at[didxb.at[t]],
                                     ss[t], add=True)
                for t in range(_DEPTH):
                    pltpu.make_async_copy(bufs[t], acc.at[didxb.at[t]],
                                          ss[t]).wait()

                    @pl.when(j + _DEPTH + t < _CPT)
                    def _(t=t):
                        pltpu.async_copy(di.at[j + _DEPTH + t],
                                         didxb.at[t], ds[t])
                        pltpu.async_copy(gsp.at[sidx_v.at[j + _DEPTH + t]],
                                         bufs[t], gs[t])

            plsc.subcore_barrier()
            for z in range(_ZC):
                r = pl.ds(rows0 + z * _CH, _CH)
                pltpu.sync_copy(acc.at[r], bufs[0])
                pltpu.sync_copy(bufs[0], out_hbm.at[kk].at[r, hl])
            plsc.subcore_barrier()

    return _prop


_prop4 = _make_prop(4)
_prop2 = _make_prop(2)


def _dis(deg_ref):
    return lax.rsqrt(deg_ref[0, :, 0:1] + deg_ref[1, :, 0:1] + 1.0)


def _tc_in(x_pad, W1, deg):
    def body(x_ref, w_ref, deg_ref, o_ref):
        dis = _dis(deg_ref)
        o_ref[0] = dis * jnp.dot(x_ref[...], w_ref[...],
                                 preferred_element_type=jnp.float32,
                                 precision=lax.Precision.DEFAULT)

    return pl.pallas_call(
        body,
        grid=(4, _RG),
        in_specs=[
            pl.BlockSpec((_RB, 256), lambda k, r: (r, 0)),
            pl.BlockSpec((256, 128), lambda k, r: (0, k)),
            pl.BlockSpec((2, _RB, 128), lambda k, r: (0, r, 0)),
        ],
        out_specs=pl.BlockSpec((1, _RB, 128), lambda k, r: (k, r, 0)),
        out_shape=jax.ShapeDtypeStruct((4, _NPAD, 128), jnp.float32),
    )(x_pad, W1, deg)


def _tc_mid(P, g, deg, b, W, kin_n, kout_n, relu):
    def body(p_ref, g_ref, deg_ref, b_ref, w_ref, o_ref):
        dis = _dis(deg_ref)
        W_all = w_ref[...]
        b_all = b_ref[...]
        acc = jnp.zeros((_RB, 128), jnp.float32)
        for kin in range(kin_n):
            t = dis * (p_ref[kin] + g_ref[kin]) + b_all[:, kin * 128:(kin + 1) * 128]
            if relu:
                t = jnp.maximum(t, 0.0)
            acc = acc + jnp.dot(t, W_all[kin * 128:(kin + 1) * 128, :],
                                preferred_element_type=jnp.float32,
                                precision=lax.Precision.DEFAULT)
        o_ref[0] = dis * acc

    return pl.pallas_call(
        body,
        grid=(kout_n, _RG),
        in_specs=[
            pl.BlockSpec((kin_n, _RB, 128), lambda k, r: (0, r, 0)),
            pl.BlockSpec((kin_n, _RB, 128), lambda k, r: (0, r, 0)),
            pl.BlockSpec((2, _RB, 128), lambda k, r: (0, r, 0)),
            pl.BlockSpec((1, kin_n * 128), lambda k, r: (0, 0)),
            pl.BlockSpec((kin_n * 128, 128), lambda k, r: (0, k)),
        ],
        out_specs=pl.BlockSpec((1, _RB, 128), lambda k, r: (k, r, 0)),
        out_shape=jax.ShapeDtypeStruct((kout_n, _NPAD, 128), jnp.float32),
    )(P, g, deg, b, W)


def _tc_out(P, g, deg, b):
    def body(p_ref, g_ref, deg_ref, b_ref, o_ref):
        dis = _dis(deg_ref)
        o_ref[...] = jax.nn.sigmoid(dis * (p_ref[0] + g_ref[0]) + b_ref[...])

    return pl.pallas_call(
        body,
        grid=(2, _RG),
        in_specs=[
            pl.BlockSpec((1, _RB, 128), lambda k, r: (k, r, 0)),
            pl.BlockSpec((1, _RB, 128), lambda k, r: (k, r, 0)),
            pl.BlockSpec((2, _RB, 128), lambda k, r: (0, r, 0)),
            pl.BlockSpec((1, 128), lambda k, r: (0, k)),
        ],
        out_specs=pl.BlockSpec((_RB, 128), lambda k, r: (r, k)),
        out_shape=jax.ShapeDtypeStruct((_NPAD, 256), jnp.float32),
    )(P, g, deg, b)


def kernel(x, edge_index, W1, b1, W2, b2, W3, b3):
    s = edge_index[0]
    d = edge_index[1]
    pad = _EPAD - _E
    s_pad = jnp.concatenate([s, jnp.zeros((pad,), jnp.int32)])
    d_pad = jnp.concatenate([d, jnp.full((pad,), _N, jnp.int32)])
    sidx = s_pad.reshape(_TILES, _CPT, _CH)
    didx = d_pad.reshape(_TILES, _CPT, _CH)
    didx_deg = d_pad.reshape(_EPAD // _CH, _CH)
    z128 = jnp.zeros((_CH, 64), jnp.float32)
    consts = jnp.stack([jnp.zeros((_CH, 128), jnp.float32),
                        jnp.ones((_CH, 128), jnp.float32)])

    deg = _deg_kernel(didx_deg, consts)
    x_pad = jnp.pad(x, ((0, _NPAD - _N), (0, 0)))
    g1 = _tc_in(x_pad, W1, deg)
    P1 = _prop4(g1, sidx, didx, z128)
    g2 = _tc_mid(P1, g1, deg, b1.reshape(1, -1), W2, 4, 4, True)
    P2 = _prop4(g2, sidx, didx, z128)
    g3 = _tc_mid(P2, g2, deg, b2.reshape(1, -1), W3, 4, 2, False)
    P3 = _prop2(g3, sidx, didx, z128)
    out = _tc_out(P3, g3, deg, b3.reshape(1, -1))
    return out[:_N]


# speedup vs baseline: 9.4798x; 1.0955x over previous
import functools

import jax
import jax.numpy as jnp
from jax import lax
from jax.experimental import pallas as pl
from jax.experimental.pallas import tpu as pltpu
from jax.experimental.pallas import tpu_sc as plsc

_N = 10000
_E = 160000
_NPAD = 10240
_EPAD = 163840
_CH = 64
_DEPTH = 8
_TILES = 16
_CPT = _EPAD // (_TILES * _CH)
_RPT = _NPAD // _TILES
_ZC = _RPT // _CH
_RB = 1024
_RG = _NPAD // _RB

_mesh = plsc.VectorSubcoreMesh(core_axis_name="c", subcore_axis_name="s")


@functools.partial(
    pl.kernel, mesh=_mesh,
    out_type=jax.ShapeDtypeStruct((2, _NPAD, 128), jnp.float32),
    scratch_types=[
        pltpu.VMEM((_CPT // 2, _CH), jnp.int32),
        pltpu.VMEM((_CH, 128), jnp.float32),
        pltpu.VMEM((_CH, 128), jnp.float32),
        pltpu.VMEM_SHARED((_NPAD, 128), jnp.float32),
    ])
def _deg_kernel(didx_hbm, consts_hbm, deg_hbm, idx_v, zbuf, obuf, acc):
    cid = lax.axis_index("c")
    sid = lax.axis_index("s")
    pltpu.sync_copy(consts_hbm.at[0], zbuf)
    pltpu.sync_copy(consts_hbm.at[1], obuf)
    w = cid * _TILES + sid
    pltpu.sync_copy(didx_hbm.at[pl.ds(w * (_CPT // 2), _CPT // 2)], idx_v)
    for z in range(_ZC):
        pltpu.sync_copy(zbuf, acc.at[pl.ds((sid * _ZC + z) * _CH, _CH)])
    plsc.subcore_barrier()

    @pl.loop(0, _CPT // 2)
    def _(j):
        pltpu.sync_copy(obuf, acc.at[idx_v.at[j]], add=True)

    plsc.subcore_barrier()
    for z in range(_ZC):
        r = pl.ds((sid * _ZC + z) * _CH, _CH)
        pltpu.sync_copy(acc.at[r], zbuf)
        pltpu.sync_copy(zbuf, deg_hbm.at[cid].at[r])


def _make_prop(nslices):
    nhalf = nslices * 2
    per_core = nhalf // 2

    @functools.partial(
        pl.kernel, mesh=_mesh,
        compiler_params=pltpu.CompilerParams(use_tc_tiling_on_sc=False),
        out_type=jax.ShapeDtypeStruct((nslices, _NPAD, 128), jnp.float32),
        scratch_types=[
            pltpu.VMEM((_CPT, _CH), jnp.int32),
            pltpu.VMEM((_DEPTH, _CH), jnp.int32),
        ] + [pltpu.VMEM((_CH, 64), jnp.float32) for _ in range(_DEPTH)] + [
            pltpu.VMEM_SHARED((_NPAD, 64), jnp.float32),
            pltpu.VMEM_SHARED((_NPAD, 64), jnp.float32),
        ] + [pltpu.SemaphoreType.DMA for _ in range(3 * _DEPTH)])
    def _prop(g_hbm, sidx_hbm, didx_hbm, z_hbm, out_hbm,
              sidx_v, didxb, *rest):
        bufs = rest[:_DEPTH]
        gsp = rest[_DEPTH]
        acc = rest[_DEPTH + 1]
        gs = rest[_DEPTH + 2:_DEPTH + 2 + _DEPTH]
        ds = rest[_DEPTH + 2 + _DEPTH:_DEPTH + 2 + 2 * _DEPTH]
        ss = rest[_DEPTH + 2 + 2 * _DEPTH:]
        cid = lax.axis_index("c")
        sid = lax.axis_index("s")
        di = didx_hbm.at[sid]
        pltpu.sync_copy(sidx_hbm.at[sid], sidx_v)
        rows0 = sid * _RPT
        for p in range(per_core):
            hs = cid * per_core + p
            kk = hs // 2
            hl = pl.ds((hs % 2) * 64, 64)
            for z in range(min(_DEPTH, _ZC)):
                r = pl.ds(rows0 + z * _CH, _CH)
                pltpu.async_copy(g_hbm.at[kk].at[r, hl], bufs[z], gs[z])
            for z in range(_ZC):
                t = z % _DEPTH
                r = pl.ds(rows0 + z * _CH, _CH)
                pltpu.make_async_copy(g_hbm.at[kk].at[r, hl],
                                      bufs[t], gs[t]).wait()
                pltpu.sync_copy(bufs[t], gsp.at[r])
                if z + _DEPTH < _ZC:
                    r2 = pl.ds(rows0 + (z + _DEPTH) * _CH, _CH)
                    pltpu.async_copy(g_hbm.at[kk].at[r2, hl], bufs[t], gs[t])
            pltpu.sync_copy(z_hbm, bufs[0])
            for z in range(_ZC):
                pltpu.sync_copy(bufs[0],
                                acc.at[pl.ds(rows0 + z * _CH, _CH)])
            plsc.subcore_barrier()

            for t in range(_DEPTH):
                pltpu.async_copy(di.at[t], didxb.at[t], ds[t])
                pltpu.async_copy(gsp.at[sidx_v.at[t]], bufs[t], gs[t])

            @pl.loop(0, _CPT, step=_DEPTH)
            def _(j):
                for t in range(_DEPTH):
                    pltpu.make_async_copy(gsp.at[sidx_v.at[j + t]],
                                          bufs[t], gs[t]).wait()
                    pltpu.make_async_copy(di.at[j + t], didxb.at[t],
                                          ds[t]).wait()
                    pltpu.async_copy(bufs[t], acc.at[didxb.at[t]],
                                     ss[t], add=True)
                for t in range(_DEPTH):
                    pltpu.make_async_copy(bufs[t], acc.at[didxb.at[t]],
                                          ss[t]).wait()

                    @pl.when(j + _DEPTH + t < _CPT)
                    def _(t=t):
                        pltpu.async_copy(di.at[j + _DEPTH + t],
                                         didxb.at[t], ds[t])
                        pltpu.async_copy(gsp.at[sidx_v.at[j + _DEPTH + t]],
                                         bufs[t], gs[t])

            plsc.subcore_barrier()
            for z in range(_ZC):
                t = z % _DEPTH
                r = pl.ds(rows0 + z * _CH, _CH)
                if z >= _DEPTH:
                    rp = pl.ds(rows0 + (z - _DEPTH) * _CH, _CH)
                    pltpu.make_async_copy(bufs[t], out_hbm.at[kk].at[rp, hl],
                                          ss[t]).wait()
                pltpu.sync_copy(acc.at[r], bufs[t])
                pltpu.async_copy(bufs[t], out_hbm.at[kk].at[r, hl], ss[t])
            for z in range(max(0, _ZC - _DEPTH), _ZC):
                t = z % _DEPTH
                r = pl.ds(rows0 + z * _CH, _CH)
                pltpu.make_async_copy(bufs[t], out_hbm.at[kk].at[r, hl],
                                      ss[t]).wait()

    return _prop


_prop4 = _make_prop(4)
_prop2 = _make_prop(2)


def _dis(deg_ref):
    return lax.rsqrt(deg_ref[0, :, 0:1] + deg_ref[1, :, 0:1] + 1.0)


def _tc_in(x_pad, W1, deg):
    def body(x_ref, w_ref, deg_ref, o_ref):
        dis = _dis(deg_ref)
        o_ref[0] = dis * jnp.dot(x_ref[...], w_ref[...],
                                 preferred_element_type=jnp.float32,
                                 precision=lax.Precision.DEFAULT)

    return pl.pallas_call(
        body,
        grid=(4, _RG),
        in_specs=[
            pl.BlockSpec((_RB, 256), lambda k, r: (r, 0)),
            pl.BlockSpec((256, 128), lambda k, r: (0, k)),
            pl.BlockSpec((2, _RB, 128), lambda k, r: (0, r, 0)),
        ],
        out_specs=pl.BlockSpec((1, _RB, 128), lambda k, r: (k, r, 0)),
        out_shape=jax.ShapeDtypeStruct((4, _NPAD, 128), jnp.float32),
    )(x_pad, W1, deg)


def _tc_mid(P, g, deg, b, W, kin_n, kout_n, relu):
    def body(p_ref, g_ref, deg_ref, b_ref, w_ref, o_ref):
        dis = _dis(deg_ref)
        W_all = w_ref[...]
        b_all = b_ref[...]
        acc = jnp.zeros((_RB, 128), jnp.float32)
        for kin in range(kin_n):
            t = dis * (p_ref[kin] + g_ref[kin]) + b_all[:, kin * 128:(kin + 1) * 128]
            if relu:
                t = jnp.maximum(t, 0.0)
            acc = acc + jnp.dot(t, W_all[kin * 128:(kin + 1) * 128, :],
                                preferred_element_type=jnp.float32,
                                precision=lax.Precision.DEFAULT)
        o_ref[0] = dis * acc

    return pl.pallas_call(
        body,
        grid=(kout_n, _RG),
        in_specs=[
            pl.BlockSpec((kin_n, _RB, 128), lambda k, r: (0, r, 0)),
            pl.BlockSpec((kin_n, _RB, 128), lambda k, r: (0, r, 0)),
            pl.BlockSpec((2, _RB, 128), lambda k, r: (0, r, 0)),
            pl.BlockSpec((1, kin_n * 128), lambda k, r: (0, 0)),
            pl.BlockSpec((kin_n * 128, 128), lambda k, r: (0, k)),
        ],
        out_specs=pl.BlockSpec((1, _RB, 128), lambda k, r: (k, r, 0)),
        out_shape=jax.ShapeDtypeStruct((kout_n, _NPAD, 128), jnp.float32),
    )(P, g, deg, b, W)


def _tc_out(P, g, deg, b):
    def body(p_ref, g_ref, deg_ref, b_ref, o_ref):
        dis = _dis(deg_ref)
        o_ref[...] = jax.nn.sigmoid(dis * (p_ref[0] + g_ref[0]) + b_ref[...])

    return pl.pallas_call(
        body,
        grid=(2, _RG),
        in_specs=[
            pl.BlockSpec((1, _RB, 128), lambda k, r: (k, r, 0)),
            pl.BlockSpec((1, _RB, 128), lambda k, r: (k, r, 0)),
            pl.BlockSpec((2, _RB, 128), lambda k, r: (0, r, 0)),
            pl.BlockSpec((1, 128), lambda k, r: (0, k)),
        ],
        out_specs=pl.BlockSpec((_RB, 128), lambda k, r: (r, k)),
        out_shape=jax.ShapeDtypeStruct((_NPAD, 256), jnp.float32),
    )(P, g, deg, b)


def kernel(x, edge_index, W1, b1, W2, b2, W3, b3):
    s = edge_index[0]
    d = edge_index[1]
    pad = _EPAD - _E
    s_pad = jnp.concatenate([s, jnp.zeros((pad,), jnp.int32)])
    d_pad = jnp.concatenate([d, jnp.full((pad,), _N, jnp.int32)])
    sidx = s_pad.reshape(_TILES, _CPT, _CH)
    didx = d_pad.reshape(_TILES, _CPT, _CH)
    didx_deg = d_pad.reshape(_EPAD // _CH, _CH)
    z128 = jnp.zeros((_CH, 64), jnp.float32)
    consts = jnp.stack([jnp.zeros((_CH, 128), jnp.float32),
                        jnp.ones((_CH, 128), jnp.float32)])

    deg = _deg_kernel(didx_deg, consts)
    x_pad = jnp.pad(x, ((0, _NPAD - _N), (0, 0)))
    g1 = _tc_in(x_pad, W1, deg)
    P1 = _prop4(g1, sidx, didx, z128)
    g2 = _tc_mid(P1, g1, deg, b1.reshape(1, -1), W2, 4, 4, True)
    P2 = _prop4(g2, sidx, didx, z128)
    g3 = _tc_mid(P2, g2, deg, b2.reshape(1, -1), W3, 4, 2, False)
    P3 = _prop2(g3, sidx, didx, z128)
    out = _tc_out(P3, g3, deg, b3.reshape(1, -1))
    return out[:_N]


# speedup vs baseline: 9.6673x; 1.0198x over previous
import functools

import jax
import jax.numpy as jnp
from jax import lax
from jax.experimental import pallas as pl
from jax.experimental.pallas import tpu as pltpu
from jax.experimental.pallas import tpu_sc as plsc

_N = 10000
_E = 160000
_NPAD = 10240
_EPAD = 163840
_CH = 64
_DEPTH = 8
_TILES = 16
_CPT = _EPAD // (_TILES * _CH)
_RPT = _NPAD // _TILES
_ZC = _RPT // _CH
_RB = 1000
_RG = _N // _RB
_DCH = 128
_DCPT = _EPAD // (32 * _DCH)

_mesh = plsc.VectorSubcoreMesh(core_axis_name="c", subcore_axis_name="s")


@functools.partial(
    pl.kernel, mesh=_mesh,
    compiler_params=pltpu.CompilerParams(use_tc_tiling_on_sc=False),
    out_type=jax.ShapeDtypeStruct((2, _NPAD, 64), jnp.float32),
    scratch_types=[
        pltpu.VMEM((_DCPT, _DCH), jnp.int32),
        pltpu.VMEM((_CH, 64), jnp.float32),
        pltpu.VMEM((_DCH, 64), jnp.float32),
        pltpu.VMEM_SHARED((_NPAD, 64), jnp.float32),
    ])
def _deg_kernel(didx_hbm, consts_hbm, deg_hbm, idx_v, zbuf, obuf, acc):
    cid = lax.axis_index("c")
    sid = lax.axis_index("s")
    pltpu.sync_copy(consts_hbm.at[pl.ds(0, _CH)], zbuf)
    pltpu.sync_copy(consts_hbm.at[pl.ds(_CH, _DCH)], obuf)
    w = cid * _TILES + sid
    pltpu.sync_copy(didx_hbm.at[pl.ds(w * _DCPT, _DCPT)], idx_v)
    for z in range(_ZC):
        pltpu.sync_copy(zbuf, acc.at[pl.ds((sid * _ZC + z) * _CH, _CH)])
    plsc.subcore_barrier()

    @pl.loop(0, _DCPT)
    def _(j):
        pltpu.sync_copy(obuf, acc.at[idx_v.at[j]], add=True)

    plsc.subcore_barrier()
    for z in range(_ZC):
        r = pl.ds((sid * _ZC + z) * _CH, _CH)
        pltpu.sync_copy(acc.at[r], zbuf)
        pltpu.sync_copy(zbuf, deg_hbm.at[cid].at[r])


def _make_prop(nslices):
    nhalf = nslices * 2
    per_core = nhalf // 2

    @functools.partial(
        pl.kernel, mesh=_mesh,
        compiler_params=pltpu.CompilerParams(use_tc_tiling_on_sc=False),
        out_type=jax.ShapeDtypeStruct((nslices, _NPAD, 128), jnp.float32),
        scratch_types=[
            pltpu.VMEM((_CPT, _CH), jnp.int32),
            pltpu.VMEM((_DEPTH, _CH), jnp.int32),
        ] + [pltpu.VMEM((_CH, 64), jnp.float32) for _ in range(_DEPTH)] + [
            pltpu.VMEM_SHARED((_NPAD, 64), jnp.float32),
            pltpu.VMEM_SHARED((_NPAD, 64), jnp.float32),
        ] + [pltpu.SemaphoreType.DMA for _ in range(3 * _DEPTH)])
    def _prop(g_hbm, sidx_hbm, didx_hbm, z_hbm, out_hbm,
              sidx_v, didxb, *rest):
        bufs = rest[:_DEPTH]
        gsp = rest[_DEPTH]
        acc = rest[_DEPTH + 1]
        gs = rest[_DEPTH + 2:_DEPTH + 2 + _DEPTH]
        ds = rest[_DEPTH + 2 + _DEPTH:_DEPTH + 2 + 2 * _DEPTH]
        ss = rest[_DEPTH + 2 + 2 * _DEPTH:]
        cid = lax.axis_index("c")
        sid = lax.axis_index("s")
        di = didx_hbm.at[sid]
        pltpu.sync_copy(sidx_hbm.at[sid], sidx_v)
        rows0 = sid * _RPT
        for p in range(per_core):
            hs = cid * per_core + p
            kk = hs // 2
            hl = pl.ds((hs % 2) * 64, 64)
            for z in range(min(_DEPTH, _ZC)):
                r = pl.ds(rows0 + z * _CH, _CH)
                pltpu.async_copy(g_hbm.at[kk].at[r, hl], bufs[z], gs[z])
            for z in range(_ZC):
                t = z % _DEPTH
                r = pl.ds(rows0 + z * _CH, _CH)
                pltpu.make_async_copy(g_hbm.at[kk].at[r, hl],
                                      bufs[t], gs[t]).wait()
                pltpu.sync_copy(bufs[t], gsp.at[r])
                if z + _DEPTH < _ZC:
                    r2 = pl.ds(rows0 + (z + _DEPTH) * _CH, _CH)
                    pltpu.async_copy(g_hbm.at[kk].at[r2, hl], bufs[t], gs[t])
            pltpu.sync_copy(z_hbm, bufs[0])
            for z in range(_ZC):
                pltpu.sync_copy(bufs[0],
                                acc.at[pl.ds(rows0 + z * _CH, _CH)])
            plsc.subcore_barrier()

            for t in range(_DEPTH):
                pltpu.async_copy(di.at[t], didxb.at[t], ds[t])
                pltpu.async_copy(gsp.at[sidx_v.at[t]], bufs[t], gs[t])

            @pl.loop(0, _CPT, step=_DEPTH)
            def _(j):
                for t in range(_DEPTH):
                    pltpu.make_async_copy(gsp.at[sidx_v.at[j + t]],
                                          bufs[t], gs[t]).wait()
                    pltpu.make_async_copy(di.at[j + t], didxb.at[t],
                                          ds[t]).wait()
                    pltpu.async_copy(bufs[t], acc.at[didxb.at[t]],
                                     ss[t], add=True)
                for t in range(_DEPTH):
                    pltpu.make_async_copy(bufs[t], acc.at[didxb.at[t]],
                                          ss[t]).wait()

                    @pl.when(j + _DEPTH + t < _CPT)
                    def _(t=t):
                        pltpu.async_copy(di.at[j + _DEPTH + t],
                                         didxb.at[t], ds[t])
                        pltpu.async_copy(gsp.at[sidx_v.at[j + _DEPTH + t]],
                                         bufs[t], gs[t])

            plsc.subcore_barrier()
            for z in range(_ZC):
                t = z % _DEPTH
                r = pl.ds(rows0 + z * _CH, _CH)
                if z >= _DEPTH:
                    rp = pl.ds(rows0 + (z - _DEPTH) * _CH, _CH)
                    pltpu.make_async_copy(bufs[t], out_hbm.at[kk].at[rp, hl],
                                          ss[t]).wait()
                pltpu.sync_copy(acc.at[r], bufs[t])
                pltpu.async_copy(bufs[t], out_hbm.at[kk].at[r, hl], ss[t])
            for z in range(max(0, _ZC - _DEPTH), _ZC):
                t = z % _DEPTH
                r = pl.ds(rows0 + z * _CH, _CH)
                pltpu.make_async_copy(bufs[t], out_hbm.at[kk].at[r, hl],
                                      ss[t]).wait()

    return _prop


_prop4 = _make_prop(4)
_prop2 = _make_prop(2)


def _dis(deg_ref):
    return lax.rsqrt(deg_ref[0, :, 0:1] + deg_ref[1, :, 0:1] + 1.0)


def _tc_in(x_pad, W1, deg):
    def body(x_ref, w_ref, deg_ref, o_ref):
        dis = _dis(deg_ref)
        o_ref[0] = dis * jnp.dot(x_ref[...], w_ref[...],
                                 preferred_element_type=jnp.float32,
                                 precision=lax.Precision.DEFAULT)

    return pl.pallas_call(
        body,
        grid=(4, _RG),
        in_specs=[
            pl.BlockSpec((_RB, 256), lambda k, r: (r, 0)),
            pl.BlockSpec((256, 128), lambda k, r: (0, k)),
            pl.BlockSpec((2, _RB, 64), lambda k, r: (0, r, 0)),
        ],
        out_specs=pl.BlockSpec((1, _RB, 128), lambda k, r: (k, r, 0)),
        out_shape=jax.ShapeDtypeStruct((4, _NPAD, 128), jnp.float32),
    )(x_pad, W1, deg)


def _tc_mid(P, g, deg, b, W, kin_n, kout_n, relu):
    def body(p_ref, g_ref, deg_ref, b_ref, w_ref, o_ref):
        dis = _dis(deg_ref)
        W_all = w_ref[...]
        b_all = b_ref[...]
        acc = jnp.zeros((_RB, 128), jnp.float32)
        for kin in range(kin_n):
            t = dis * (p_ref[kin] + g_ref[kin]) + b_all[:, kin * 128:(kin + 1) * 128]
            if relu:
                t = jnp.maximum(t, 0.0)
            acc = acc + jnp.dot(t, W_all[kin * 128:(kin + 1) * 128, :],
                                preferred_element_type=jnp.float32,
                                precision=lax.Precision.DEFAULT)
        o_ref[0] = dis * acc

    return pl.pallas_call(
        body,
        grid=(kout_n, _RG),
        in_specs=[
            pl.BlockSpec((kin_n, _RB, 128), lambda k, r: (0, r, 0)),
            pl.BlockSpec((kin_n, _RB, 128), lambda k, r: (0, r, 0)),
            pl.BlockSpec((2, _RB, 64), lambda k, r: (0, r, 0)),
            pl.BlockSpec((1, kin_n * 128), lambda k, r: (0, 0)),
            pl.BlockSpec((kin_n * 128, 128), lambda k, r: (0, k)),
        ],
        out_specs=pl.BlockSpec((1, _RB, 128), lambda k, r: (k, r, 0)),
        out_shape=jax.ShapeDtypeStruct((kout_n, _NPAD, 128), jnp.float32),
    )(P, g, deg, b, W)


def _tc_out(P, g, deg, b):
    def body(p_ref, g_ref, deg_ref, b_ref, o_ref):
        dis = _dis(deg_ref)
        o_ref[...] = jax.nn.sigmoid(dis * (p_ref[0] + g_ref[0]) + b_ref[...])

    return pl.pallas_call(
        body,
        grid=(2, _RG),
        in_specs=[
            pl.BlockSpec((1, _RB, 128), lambda k, r: (k, r, 0)),
            pl.BlockSpec((1, _RB, 128), lambda k, r: (k, r, 0)),
            pl.BlockSpec((2, _RB, 64), lambda k, r: (0, r, 0)),
            pl.BlockSpec((1, 128), lambda k, r: (0, k)),
        ],
        out_specs=pl.BlockSpec((_RB, 128), lambda k, r: (r, k)),
        out_shape=jax.ShapeDtypeStruct((_N, 256), jnp.float32),
    )(P, g, deg, b)


def kernel(x, edge_index, W1, b1, W2, b2, W3, b3):
    s = edge_index[0]
    d = edge_index[1]
    pad = _EPAD - _E
    s_pad = jnp.concatenate([s, jnp.zeros((pad,), jnp.int32)])
    d_pad = jnp.concatenate([d, jnp.full((pad,), _N, jnp.int32)])
    sidx = s_pad.reshape(_TILES, _CPT, _CH)
    didx = d_pad.reshape(_TILES, _CPT, _CH)
    didx_deg = d_pad.reshape(_EPAD // _DCH, _DCH)
    z128 = jnp.zeros((_CH, 64), jnp.float32)
    consts = jnp.concatenate([jnp.zeros((_CH, 64), jnp.float32),
                              jnp.ones((_DCH, 64), jnp.float32)])

    deg = _deg_kernel(didx_deg, consts)
    g1 = _tc_in(x, W1, deg)
    P1 = _prop4(g1, sidx, didx, z128)
    g2 = _tc_mid(P1, g1, deg, b1.reshape(1, -1), W2, 4, 4, True)
    P2 = _prop4(g2, sidx, didx, z128)
    g3 = _tc_mid(P2, g2, deg, b2.reshape(1, -1), W3, 4, 2, False)
    P3 = _prop2(g3, sidx, didx, z128)
    return _tc_out(P3, g3, deg, b3.reshape(1, -1))
